# baseline (speedup 1.0000x reference)
import functools

import jax
import jax.numpy as jnp
from jax import lax
from jax.experimental import pallas as pl
from jax.experimental.pallas import tpu as pltpu
from jax.experimental.pallas import tpu_sc as plsc

_E = 16
_D = 1024
_DFF = 512
_S = 2048
_TM = 128
_NT = 48
_PAD = _NT * _TM
_NW = 32
_RPW = _PAD // _NW
_TPW = _S // _NW


def _router_body(x_ref, gate_ref, pos_ref, prob_ref, te_ref):
    x = x_ref[:]
    scores = lax.dot_general(gate_ref[:], x, (((1,), (1,)), ((), ())),
                             preferred_element_type=jnp.float32)
    ie = lax.broadcasted_iota(jnp.int32, (_E, _S), 0)
    m1 = jnp.max(scores, axis=0, keepdims=True)
    i1 = jnp.min(jnp.where(scores == m1, ie, _E), axis=0, keepdims=True)
    masked = jnp.where(ie == i1, -jnp.inf, scores)
    m2 = jnp.max(masked, axis=0, keepdims=True)
    i2 = jnp.min(jnp.where(masked == m2, ie, _E), axis=0, keepdims=True)
    t = jnp.exp(m2 - m1)
    p1 = 1.0 / (1.0 + t)
    p2 = 1.0 - p1
    maskf = jnp.logical_or(ie == i1, ie == i2).astype(jnp.float32)

    lu = (lax.broadcasted_iota(jnp.int32, (_TM, _TM), 0)
          <= lax.broadcasted_iota(jnp.int32, (_TM, _TM), 1)).astype(jnp.float32)
    chunks = []
    carry = jnp.zeros((_E, 1), jnp.float32)
    for c in range(_S // _TM):
        mc = maskf[:, c * _TM:(c + 1) * _TM]
        cs = lax.dot_general(mc, lu, (((1,), (0,)), ((), ())),
                             preferred_element_type=jnp.float32)
        chunks.append(cs + carry)
        carry = carry + cs[:, _TM - 1:_TM]
    incl = jnp.concatenate(chunks, axis=1)

    cnt = incl[:, _S - 1:_S]
    cntp = jnp.ceil(cnt * (1.0 / _TM)) * float(_TM)
    slm = (lax.broadcasted_iota(jnp.int32, (_E, _E), 1)
           < lax.broadcasted_iota(jnp.int32, (_E, _E), 0)).astype(jnp.float32)
    offp = lax.dot_general(slm, cntp, (((1,), (0,)), ((), ())),
                           preferred_element_type=jnp.float32)
    posmat = offp + incl - 1.0
    on1 = (ie == i1).astype(jnp.float32)
    on2 = (ie == i2).astype(jnp.float32)
    pos1 = jnp.sum(on1 * posmat, axis=0, keepdims=True)
    pos2 = jnp.sum(on2 * posmat, axis=0, keepdims=True)
    pos_ref[:] = jnp.concatenate([pos1, pos2], axis=0).astype(jnp.int32)
    prob_ref[:] = jnp.concatenate([p1, p2], axis=0)

    ends = offp + cntp
    jv = (lax.broadcasted_iota(jnp.int32, (_E, _NT), 1) * _TM).astype(jnp.float32)
    tef = jnp.sum((ends <= jv).astype(jnp.float32), axis=0, keepdims=True)
    te_ref[:] = jnp.minimum(tef, float(_E - 1)).astype(jnp.int32)


def _router(x2d, gate_w, interpret=False):
    return pl.pallas_call(
        _router_body,
        out_shape=(
            jax.ShapeDtypeStruct((2, _S), jnp.int32),
            jax.ShapeDtypeStruct((2, _S), jnp.float32),
            jax.ShapeDtypeStruct((1, _NT), jnp.int32),
        ),
        interpret=interpret,
    )(x2d, gate_w)


def _scatter_body(pos_hbm, prob_hbm, src_hbm, ps_hbm, pos_v, prob_v, srcb, psb):
    cid = lax.axis_index("c")
    sid = lax.axis_index("s")

    @pl.when(jnp.logical_and(cid == 0, sid == 0))
    def _():
        pltpu.sync_copy(pos_hbm, pos_v)
        pltpu.sync_copy(prob_hbm, prob_v)
        zi = jnp.zeros((16,), jnp.int32)
        zf = jnp.zeros((16,), jnp.float32)

        def init(i, c):
            srcb[pl.ds(i * 16, 16)] = zi
            psb[pl.ds(i * 16, 16)] = zf
            return c

        lax.fori_loop(0, _PAD // 16, init, 0)
        lane = lax.iota(jnp.int32, 16)

        def body(i, c):
            base = i * 16
            p = pos_v[pl.ds(base, 16)]
            pr = prob_v[pl.ds(base, 16)]
            tok = lane + lax.rem(base, _S)
            plsc.store_scatter(srcb, [p], tok)
            plsc.store_scatter(psb, [p], pr)
            return c

        lax.fori_loop(0, (2 * _S) // 16, body, 0)
        pltpu.sync_copy(srcb, src_hbm)
        pltpu.sync_copy(psb, ps_hbm)


_GCH = 32
_GNB = 3


def _gather_body(src_hbm, x_hbm, xs_hbm, idx_v, rows_v, sem0, sem1, sem2):
    wid = lax.axis_index("s") * 2 + lax.axis_index("c")
    base = wid * _RPW
    nch = _RPW // _GCH
    sems = (sem0, sem1, sem2)
    for c in range(nch):
        pltpu.sync_copy(src_hbm.at[pl.ds(base + c * _GCH, _GCH)], idx_v.at[c])
    cps = {}
    for c in range(nch):
        b = c % _GNB
        if c >= _GNB:
            cps[c - _GNB].wait()
            pltpu.sync_copy(rows_v.at[b],
                            xs_hbm.at[pl.ds(base + (c - _GNB) * _GCH, _GCH)])
        cps[c] = pltpu.async_copy(
            x_hbm.at[idx_v.at[c]], rows_v.at[b], sems[b])
    for c in range(nch - _GNB, nch):
        b = c % _GNB
        cps[c].wait()
        pltpu.sync_copy(rows_v.at[b], xs_hbm.at[pl.ds(base + c * _GCH, _GCH)])


def _grouped_body(te_ref, x_ref, wg_ref, wu_ref, wd_ref, pb_ref, o_ref,
                  wgb, wub, wdb):
    j = pl.program_id(0)
    changed = jnp.logical_or(
        j == 0, te_ref[j] != te_ref[jnp.maximum(j - 1, 0)])

    @pl.when(changed)
    def _convert():
        wgb[:] = wg_ref[0].astype(jnp.bfloat16)
        wub[:] = wu_ref[0].astype(jnp.bfloat16)
        wdb[:] = wd_ref[0].astype(jnp.bfloat16)

    x = x_ref[:].astype(jnp.bfloat16)
    g = lax.dot_general(x, wgb[:], (((1,), (1,)), ((), ())),
                        preferred_element_type=jnp.float32)
    u = lax.dot_general(x, wub[:], (((1,), (1,)), ((), ())),
                        preferred_element_type=jnp.float32)
    h = (g * (1.0 / (1.0 + jnp.exp(-g)))) * u
    out = lax.dot_general(h.astype(jnp.bfloat16), wdb[:],
                          (((1,), (1,)), ((), ())),
                          preferred_element_type=jnp.float32)
    o_ref[:] = out * pb_ref[:]


def _grouped(te, xs, Wg, Wu, Wd, ps, interpret=False):
    grid_spec = pltpu.PrefetchScalarGridSpec(
        num_scalar_prefetch=1,
        grid=(_NT,),
        in_specs=[
            pl.BlockSpec((_TM, _D), lambda j, te: (j, 0)),
            pl.BlockSpec((1, _DFF, _D), lambda j, te: (te[j], 0, 0)),
            pl.BlockSpec((1, _DFF, _D), lambda j, te: (te[j], 0, 0)),
            pl.BlockSpec((1, _D, _DFF), lambda j, te: (te[j], 0, 0)),
            pl.BlockSpec((_TM, 1), lambda j, te: (j, 0)),
        ],
        out_specs=pl.BlockSpec((_TM, _D), lambda j, te: (j, 0)),
        scratch_shapes=[
            pltpu.VMEM((_DFF, _D), jnp.bfloat16),
            pltpu.VMEM((_DFF, _D), jnp.bfloat16),
            pltpu.VMEM((_D, _DFF), jnp.bfloat16),
        ],
    )
    return pl.pallas_call(
        _grouped_body,
        grid_spec=grid_spec,
        out_shape=jax.ShapeDtypeStruct((_PAD, _D), jnp.float32),
        interpret=interpret,
    )(te, xs, Wg, Wu, Wd, ps)


_CCH = 32


def _combine_body(ys_hbm, pos_hbm, y_hbm, i1_v, i2_v, r1_v, r2_v, sem1, sem2):
    wid = lax.axis_index("s") * 2 + lax.axis_index("c")
    base = wid * _TPW
    for c in range(_TPW // _CCH):
        tb = base + c * _CCH
        pltpu.sync_copy(pos_hbm.at[0, pl.ds(tb, _CCH)], i1_v)
        pltpu.sync_copy(pos_hbm.at[1, pl.ds(tb, _CCH)], i2_v)
        cp1 = pltpu.async_copy(ys_hbm.at[i1_v], r1_v, sem1)
        cp2 = pltpu.async_copy(ys_hbm.at[i2_v], r2_v, sem2)
        cp1.wait()
        cp2.wait()

        def addt(t, carry):
            for k in range(_D // 16):
                sl = pl.ds(k * 16, 16)
                r1_v[t, sl] = r1_v[t, sl] + r2_v[t, sl]
            return carry

        lax.fori_loop(0, _CCH, addt, 0)
        pltpu.sync_copy(r1_v, y_hbm.at[pl.ds(tb, _CCH)])


@functools.cache
def _sc_kernels():
    mesh = plsc.VectorSubcoreMesh(core_axis_name="c", subcore_axis_name="s")
    no_layout = pltpu.CompilerParams(needs_layout_passes=False)
    scatter = pl.kernel(
        _scatter_body,
        compiler_params=no_layout,
        out_type=(
            jax.ShapeDtypeStruct((_PAD,), jnp.int32),
            jax.ShapeDtypeStruct((_PAD,), jnp.float32),
        ),
        mesh=mesh,
        scratch_types=[
            pltpu.VMEM((2 * _S,), jnp.int32),
            pltpu.VMEM((2 * _S,), jnp.float32),
            pltpu.VMEM((_PAD,), jnp.int32),
            pltpu.VMEM((_PAD,), jnp.float32),
        ],
    )
    gather = pl.kernel(
        _gather_body,
        out_type=jax.ShapeDtypeStruct((_PAD, _D), jnp.float32),
        mesh=mesh,
        scratch_types=[
            pltpu.VMEM((_RPW // _GCH, _GCH), jnp.int32),
            pltpu.VMEM((_GNB, _GCH, _D), jnp.float32),
            pltpu.SemaphoreType.DMA,
            pltpu.SemaphoreType.DMA,
            pltpu.SemaphoreType.DMA,
        ],
    )
    combine = pl.kernel(
        _combine_body,
        out_type=jax.ShapeDtypeStruct((_S, _D), jnp.float32),
        mesh=mesh,
        scratch_types=[
            pltpu.VMEM((_CCH,), jnp.int32),
            pltpu.VMEM((_CCH,), jnp.int32),
            pltpu.VMEM((_CCH, _D), jnp.float32),
            pltpu.VMEM((_CCH, _D), jnp.float32),
            pltpu.SemaphoreType.DMA,
            pltpu.SemaphoreType.DMA,
        ],
    )
    return scatter, gather, combine


def kernel(x, gate_w, Wg, Wu, Wd):
    b, s, d = x.shape
    scatter, gather, combine = _sc_kernels()
    x2d = x.reshape(s, d)
    pos12, probs12, te = _router(x2d, gate_w)
    src, ps = scatter(pos12.reshape(2 * _S), probs12.reshape(2 * _S))
    xs = gather(src, x2d)
    ys = _grouped(te.reshape(_NT), xs, Wg, Wu, Wd, ps.reshape(_PAD, 1))
    y = combine(ys, pos12)
    return y.reshape(b, s, d)


# speedup vs baseline: 1.0006x; 1.0006x over previous
import functools

import jax
import jax.numpy as jnp
from jax import lax
from jax.experimental import pallas as pl
from jax.experimental.pallas import tpu as pltpu
from jax.experimental.pallas import tpu_sc as plsc

_E = 16
_D = 1024
_DFF = 512
_S = 2048
_TM = 128
_NT = 48
_PAD = _NT * _TM
_NW = 32
_RPW = _PAD // _NW
_TPW = _S // _NW


def _router_body(x_ref, gate_ref, pos_ref, prob_ref, te_ref):
    x = x_ref[:]
    scores = lax.dot_general(gate_ref[:], x, (((1,), (1,)), ((), ())),
                             preferred_element_type=jnp.float32)
    ie = lax.broadcasted_iota(jnp.int32, (_E, _S), 0)
    m1 = jnp.max(scores, axis=0, keepdims=True)
    i1 = jnp.min(jnp.where(scores == m1, ie, _E), axis=0, keepdims=True)
    masked = jnp.where(ie == i1, -jnp.inf, scores)
    m2 = jnp.max(masked, axis=0, keepdims=True)
    i2 = jnp.min(jnp.where(masked == m2, ie, _E), axis=0, keepdims=True)
    t = jnp.exp(m2 - m1)
    p1 = 1.0 / (1.0 + t)
    p2 = 1.0 - p1
    maskf = jnp.logical_or(ie == i1, ie == i2).astype(jnp.float32)

    lu = (lax.broadcasted_iota(jnp.int32, (_TM, _TM), 0)
          <= lax.broadcasted_iota(jnp.int32, (_TM, _TM), 1)).astype(jnp.float32)
    chunks = []
    carry = jnp.zeros((_E, 1), jnp.float32)
    for c in range(_S // _TM):
        mc = maskf[:, c * _TM:(c + 1) * _TM]
        cs = lax.dot_general(mc, lu, (((1,), (0,)), ((), ())),
                             preferred_element_type=jnp.float32)
        chunks.append(cs + carry)
        carry = carry + cs[:, _TM - 1:_TM]
    incl = jnp.concatenate(chunks, axis=1)

    cnt = incl[:, _S - 1:_S]
    cntp = jnp.ceil(cnt * (1.0 / _TM)) * float(_TM)
    slm = (lax.broadcasted_iota(jnp.int32, (_E, _E), 1)
           < lax.broadcasted_iota(jnp.int32, (_E, _E), 0)).astype(jnp.float32)
    offp = lax.dot_general(slm, cntp, (((1,), (0,)), ((), ())),
                           preferred_element_type=jnp.float32)
    posmat = offp + incl - 1.0
    on1 = (ie == i1).astype(jnp.float32)
    on2 = (ie == i2).astype(jnp.float32)
    pos1 = jnp.sum(on1 * posmat, axis=0, keepdims=True)
    pos2 = jnp.sum(on2 * posmat, axis=0, keepdims=True)
    pos_ref[:] = jnp.concatenate([pos1, pos2], axis=0).astype(jnp.int32)
    prob_ref[:] = jnp.concatenate([p1, p2], axis=0)

    ends = offp + cntp
    jv = (lax.broadcasted_iota(jnp.int32, (_E, _NT), 1) * _TM).astype(jnp.float32)
    tef = jnp.sum((ends <= jv).astype(jnp.float32), axis=0, keepdims=True)
    te_ref[:] = jnp.minimum(tef, float(_E - 1)).astype(jnp.int32)


def _router(x2d, gate_w, interpret=False):
    return pl.pallas_call(
        _router_body,
        out_shape=(
            jax.ShapeDtypeStruct((2, _S), jnp.int32),
            jax.ShapeDtypeStruct((2, _S), jnp.float32),
            jax.ShapeDtypeStruct((1, _NT), jnp.int32),
        ),
        interpret=interpret,
    )(x2d, gate_w)


def _scatter_body(pos_hbm, prob_hbm, src_hbm, ps_hbm, pos_v, prob_v, srcb, psb):
    cid = lax.axis_index("c")
    sid = lax.axis_index("s")

    @pl.when(jnp.logical_and(cid == 0, sid == 0))
    def _():
        pltpu.sync_copy(pos_hbm, pos_v)
        pltpu.sync_copy(prob_hbm, prob_v)
        zi = jnp.zeros((16,), jnp.int32)
        zf = jnp.zeros((16,), jnp.float32)

        def init(i, c):
            srcb[pl.ds(i * 16, 16)] = zi
            psb[pl.ds(i * 16, 16)] = zf
            return c

        lax.fori_loop(0, _PAD // 16, init, 0)
        lane = lax.iota(jnp.int32, 16)

        def body(i, c):
            base = i * 16
            p = pos_v[pl.ds(base, 16)]
            pr = prob_v[pl.ds(base, 16)]
            tok = lane + lax.rem(base, _S)
            plsc.store_scatter(srcb, [p], tok)
            plsc.store_scatter(psb, [p], pr)
            return c

        lax.fori_loop(0, (2 * _S) // 16, body, 0)
        pltpu.sync_copy(srcb, src_hbm)
        pltpu.sync_copy(psb, ps_hbm)


_GCH = 32
_GNB = 3


def _gather_body(src_hbm, x_hbm, xs_hbm, idx_v, rows0, rows1, rows2,
                 sem0, sem1, sem2):
    wid = lax.axis_index("s") * 2 + lax.axis_index("c")
    base = wid * _RPW
    nch = _RPW // _GCH
    bufs = (rows0, rows1, rows2)
    sems = (sem0, sem1, sem2)
    for c in range(nch):
        pltpu.sync_copy(src_hbm.at[pl.ds(base + c * _GCH, _GCH)], idx_v.at[c])
    cps = {}
    for c in range(nch):
        b = c % _GNB
        if c >= _GNB:
            cps[c - _GNB].wait()
            pltpu.sync_copy(bufs[b],
                            xs_hbm.at[pl.ds(base + (c - _GNB) * _GCH, _GCH)])
        cps[c] = pltpu.async_copy(x_hbm.at[idx_v.at[c]], bufs[b], sems[b])
    for c in range(nch - _GNB, nch):
        b = c % _GNB
        cps[c].wait()
        pltpu.sync_copy(bufs[b], xs_hbm.at[pl.ds(base + c * _GCH, _GCH)])


def _grouped_body(te_ref, x_ref, wg_ref, wu_ref, wd_ref, pb_ref, o_ref,
                  wgb, wub, wdb, xb, hb):
    j = pl.program_id(0)
    changed = jnp.logical_or(
        j == 0, te_ref[j] != te_ref[jnp.maximum(j - 1, 0)])

    @pl.when(changed)
    def _convert():
        wgb[:] = wg_ref[0].astype(jnp.bfloat16)
        wub[:] = wu_ref[0].astype(jnp.bfloat16)
        wdb[:] = wd_ref[0].astype(jnp.bfloat16)

    xb[:] = x_ref[:].astype(jnp.bfloat16)
    x = xb[:]
    g = lax.dot_general(x, wgb[:], (((1,), (1,)), ((), ())),
                        preferred_element_type=jnp.float32)
    u = lax.dot_general(x, wub[:], (((1,), (1,)), ((), ())),
                        preferred_element_type=jnp.float32)
    hb[:] = ((g * (1.0 / (1.0 + jnp.exp(-g)))) * u).astype(jnp.bfloat16)
    out = lax.dot_general(hb[:], wdb[:], (((1,), (1,)), ((), ())),
                          preferred_element_type=jnp.float32)
    o_ref[:] = out * pb_ref[:]


def _grouped(te, xs, Wg, Wu, Wd, ps, interpret=False):
    grid_spec = pltpu.PrefetchScalarGridSpec(
        num_scalar_prefetch=1,
        grid=(_NT,),
        in_specs=[
            pl.BlockSpec((_TM, _D), lambda j, te: (j, 0)),
            pl.BlockSpec((1, _DFF, _D), lambda j, te: (te[j], 0, 0)),
            pl.BlockSpec((1, _DFF, _D), lambda j, te: (te[j], 0, 0)),
            pl.BlockSpec((1, _D, _DFF), lambda j, te: (te[j], 0, 0)),
            pl.BlockSpec((_TM, 1), lambda j, te: (j, 0)),
        ],
        out_specs=pl.BlockSpec((_TM, _D), lambda j, te: (j, 0)),
        scratch_shapes=[
            pltpu.VMEM((_DFF, _D), jnp.bfloat16),
            pltpu.VMEM((_DFF, _D), jnp.bfloat16),
            pltpu.VMEM((_D, _DFF), jnp.bfloat16),
            pltpu.VMEM((_TM, _D), jnp.bfloat16),
            pltpu.VMEM((_TM, _DFF), jnp.bfloat16),
        ],
    )
    return pl.pallas_call(
        _grouped_body,
        grid_spec=grid_spec,
        out_shape=jax.ShapeDtypeStruct((_PAD, _D), jnp.float32),
        interpret=interpret,
    )(te, xs, Wg, Wu, Wd, ps)


_CCH = 32


def _combine_body(ys_hbm, pos_hbm, y_hbm, i1_v, i2_v, r1_v, r2_v, sem1, sem2):
    wid = lax.axis_index("s") * 2 + lax.axis_index("c")
    base = wid * _TPW
    for c in range(_TPW // _CCH):
        tb = base + c * _CCH
        pltpu.sync_copy(pos_hbm.at[0, pl.ds(tb, _CCH)], i1_v)
        pltpu.sync_copy(pos_hbm.at[1, pl.ds(tb, _CCH)], i2_v)
        cp1 = pltpu.async_copy(ys_hbm.at[i1_v], r1_v, sem1)
        cp2 = pltpu.async_copy(ys_hbm.at[i2_v], r2_v, sem2)
        cp1.wait()
        cp2.wait()

        def addt(t, carry):
            for k in range(_D // 16):
                sl = pl.ds(k * 16, 16)
                r1_v[t, sl] = r1_v[t, sl] + r2_v[t, sl]
            return carry

        lax.fori_loop(0, _CCH, addt, 0)
        pltpu.sync_copy(r1_v, y_hbm.at[pl.ds(tb, _CCH)])


@functools.cache
def _sc_kernels():
    mesh = plsc.VectorSubcoreMesh(core_axis_name="c", subcore_axis_name="s")
    no_layout = pltpu.CompilerParams(needs_layout_passes=False)
    scatter = pl.kernel(
        _scatter_body,
        compiler_params=no_layout,
        out_type=(
            jax.ShapeDtypeStruct((_PAD,), jnp.int32),
            jax.ShapeDtypeStruct((_PAD,), jnp.float32),
        ),
        mesh=mesh,
        scratch_types=[
            pltpu.VMEM((2 * _S,), jnp.int32),
            pltpu.VMEM((2 * _S,), jnp.float32),
            pltpu.VMEM((_PAD,), jnp.int32),
            pltpu.VMEM((_PAD,), jnp.float32),
        ],
    )
    gather = pl.kernel(
        _gather_body,
        out_type=jax.ShapeDtypeStruct((_PAD, _D), jnp.float32),
        mesh=mesh,
        scratch_types=[
            pltpu.VMEM((_RPW // _GCH, _GCH), jnp.int32),
            pltpu.VMEM((_GCH, _D), jnp.float32),
            pltpu.VMEM((_GCH, _D), jnp.float32),
            pltpu.VMEM((_GCH, _D), jnp.float32),
            pltpu.SemaphoreType.DMA,
            pltpu.SemaphoreType.DMA,
            pltpu.SemaphoreType.DMA,
        ],
    )
    combine = pl.kernel(
        _combine_body,
        out_type=jax.ShapeDtypeStruct((_S, _D), jnp.float32),
        mesh=mesh,
        scratch_types=[
            pltpu.VMEM((_CCH,), jnp.int32),
            pltpu.VMEM((_CCH,), jnp.int32),
            pltpu.VMEM((_CCH, _D), jnp.float32),
            pltpu.VMEM((_CCH, _D), jnp.float32),
            pltpu.SemaphoreType.DMA,
            pltpu.SemaphoreType.DMA,
        ],
    )
    return scatter, gather, combine


def kernel(x, gate_w, Wg, Wu, Wd):
    b, s, d = x.shape
    scatter, gather, combine = _sc_kernels()
    x2d = x.reshape(s, d)
    pos12, probs12, te = _router(x2d, gate_w)
    src, ps = scatter(pos12.reshape(2 * _S), probs12.reshape(2 * _S))
    xs = gather(src, x2d)
    ys = _grouped(te.reshape(_NT), xs, Wg, Wu, Wd, ps.reshape(_PAD, 1))
    y = combine(ys, pos12)
    return y.reshape(b, s, d)


# speedup vs baseline: 1.0204x; 1.0198x over previous
import functools

import jax
import jax.numpy as jnp
from jax import lax
from jax.experimental import pallas as pl
from jax.experimental.pallas import tpu as pltpu
from jax.experimental.pallas import tpu_sc as plsc

_E = 16
_D = 1024
_DFF = 512
_S = 2048
_TM = 128
_NT = 48
_PAD = _NT * _TM
_NW = 32
_RPW = _PAD // _NW
_TPW = _S // _NW


def _router_body(x_ref, gate_ref, pos_ref, prob_ref, te_ref):
    x = x_ref[:]
    scores = lax.dot_general(gate_ref[:], x, (((1,), (1,)), ((), ())),
                             preferred_element_type=jnp.float32)
    ie = lax.broadcasted_iota(jnp.int32, (_E, _S), 0)
    m1 = jnp.max(scores, axis=0, keepdims=True)
    i1 = jnp.min(jnp.where(scores == m1, ie, _E), axis=0, keepdims=True)
    masked = jnp.where(ie == i1, -jnp.inf, scores)
    m2 = jnp.max(masked, axis=0, keepdims=True)
    i2 = jnp.min(jnp.where(masked == m2, ie, _E), axis=0, keepdims=True)
    t = jnp.exp(m2 - m1)
    p1 = 1.0 / (1.0 + t)
    p2 = 1.0 - p1
    maskf = jnp.logical_or(ie == i1, ie == i2).astype(jnp.float32)

    lu = (lax.broadcasted_iota(jnp.int32, (_TM, _TM), 0)
          <= lax.broadcasted_iota(jnp.int32, (_TM, _TM), 1)).astype(jnp.float32)
    chunks = []
    carry = jnp.zeros((_E, 1), jnp.float32)
    for c in range(_S // _TM):
        mc = maskf[:, c * _TM:(c + 1) * _TM]
        cs = lax.dot_general(mc, lu, (((1,), (0,)), ((), ())),
                             preferred_element_type=jnp.float32)
        chunks.append(cs + carry)
        carry = carry + cs[:, _TM - 1:_TM]
    incl = jnp.concatenate(chunks, axis=1)

    cnt = incl[:, _S - 1:_S]
    cntp = jnp.ceil(cnt * (1.0 / _TM)) * float(_TM)
    slm = (lax.broadcasted_iota(jnp.int32, (_E, _E), 1)
           < lax.broadcasted_iota(jnp.int32, (_E, _E), 0)).astype(jnp.float32)
    offp = lax.dot_general(slm, cntp, (((1,), (0,)), ((), ())),
                           preferred_element_type=jnp.float32)
    posmat = offp + incl - 1.0
    on1 = (ie == i1).astype(jnp.float32)
    on2 = (ie == i2).astype(jnp.float32)
    pos1 = jnp.sum(on1 * posmat, axis=0, keepdims=True)
    pos2 = jnp.sum(on2 * posmat, axis=0, keepdims=True)
    pos_ref[:] = jnp.concatenate([pos1, pos2], axis=0).astype(jnp.int32)
    prob_ref[:] = jnp.concatenate([p1, p2], axis=0)

    ends = offp + cntp
    jv = (lax.broadcasted_iota(jnp.int32, (_E, _NT), 1) * _TM).astype(jnp.float32)
    tef = jnp.sum((ends <= jv).astype(jnp.float32), axis=0, keepdims=True)
    te_ref[:] = jnp.minimum(tef, float(_E - 1)).astype(jnp.int32)


def _router(x2d, gate_w, interpret=False):
    return pl.pallas_call(
        _router_body,
        out_shape=(
            jax.ShapeDtypeStruct((2, _S), jnp.int32),
            jax.ShapeDtypeStruct((2, _S), jnp.float32),
            jax.ShapeDtypeStruct((1, _NT), jnp.int32),
        ),
        interpret=interpret,
    )(x2d, gate_w)


def _scatter_body(pos_hbm, prob_hbm, src_hbm, ps_hbm, pos_v, prob_v, srcb, psb):
    cid = lax.axis_index("c")
    sid = lax.axis_index("s")

    @pl.when(jnp.logical_and(cid == 0, sid == 0))
    def _():
        pltpu.sync_copy(pos_hbm, pos_v)
        pltpu.sync_copy(prob_hbm, prob_v)
        zi = jnp.zeros((16,), jnp.int32)
        zf = jnp.zeros((16,), jnp.float32)

        def init(i, c):
            srcb[pl.ds(i * 16, 16)] = zi
            psb[pl.ds(i * 16, 16)] = zf
            return c

        lax.fori_loop(0, _PAD // 16, init, 0)
        lane = lax.iota(jnp.int32, 16)

        def body(i, c):
            base = i * 16
            p = pos_v[pl.ds(base, 16)]
            pr = prob_v[pl.ds(base, 16)]
            tok = lane + lax.rem(base, _S)
            plsc.store_scatter(srcb, [p], tok)
            plsc.store_scatter(psb, [p], pr)
            return c

        lax.fori_loop(0, (2 * _S) // 16, body, 0)
        pltpu.sync_copy(srcb, src_hbm)
        pltpu.sync_copy(psb, ps_hbm)


_GCH = 32
_GNB = 3


def _gather_body(src_hbm, x_hbm, xs_hbm, i1_v, i2_v, r1_v, r2_v, sem1, sem2):
    wid = lax.axis_index("s") * 2 + lax.axis_index("c")
    base = wid * _RPW
    for c in range(_RPW // (2 * _GCH)):
        cb = base + c * 2 * _GCH
        pltpu.sync_copy(src_hbm.at[pl.ds(cb, _GCH)], i1_v)
        pltpu.sync_copy(src_hbm.at[pl.ds(cb + _GCH, _GCH)], i2_v)
        cp1 = pltpu.async_copy(x_hbm.at[i1_v], r1_v, sem1)
        cp2 = pltpu.async_copy(x_hbm.at[i2_v], r2_v, sem2)
        cp1.wait()
        cp2.wait()
        pltpu.sync_copy(r1_v, xs_hbm.at[pl.ds(cb, _GCH)])
        pltpu.sync_copy(r2_v, xs_hbm.at[pl.ds(cb + _GCH, _GCH)])


def _grouped_body(te_ref, x_ref, wg_ref, wu_ref, wd_ref, pb_ref, o_ref):
    x = x_ref[:]
    g = lax.dot_general(x, wg_ref[0], (((1,), (1,)), ((), ())),
                        preferred_element_type=jnp.float32,
                        precision=lax.Precision.DEFAULT)
    u = lax.dot_general(x, wu_ref[0], (((1,), (1,)), ((), ())),
                        preferred_element_type=jnp.float32,
                        precision=lax.Precision.DEFAULT)
    h = (g * (1.0 / (1.0 + jnp.exp(-g)))) * u
    out = lax.dot_general(h, wd_ref[0], (((1,), (1,)), ((), ())),
                          preferred_element_type=jnp.float32,
                          precision=lax.Precision.DEFAULT)
    o_ref[:] = out * pb_ref[:]


def _grouped(te, xs, Wg, Wu, Wd, ps, interpret=False):
    grid_spec = pltpu.PrefetchScalarGridSpec(
        num_scalar_prefetch=1,
        grid=(_NT,),
        in_specs=[
            pl.BlockSpec((_TM, _D), lambda j, te: (j, 0)),
            pl.BlockSpec((1, _DFF, _D), lambda j, te: (te[j], 0, 0)),
            pl.BlockSpec((1, _DFF, _D), lambda j, te: (te[j], 0, 0)),
            pl.BlockSpec((1, _D, _DFF), lambda j, te: (te[j], 0, 0)),
            pl.BlockSpec((_TM, 1), lambda j, te: (j, 0)),
        ],
        out_specs=pl.BlockSpec((_TM, _D), lambda j, te: (j, 0)),
    )
    return pl.pallas_call(
        _grouped_body,
        grid_spec=grid_spec,
        out_shape=jax.ShapeDtypeStruct((_PAD, _D), jnp.float32),
        interpret=interpret,
    )(te, xs, Wg, Wu, Wd, ps)


_CCH = 32


def _combine_body(ys_hbm, pos_hbm, y_hbm, i1_v, i2_v, r1_v, r2_v, sem1, sem2):
    wid = lax.axis_index("s") * 2 + lax.axis_index("c")
    base = wid * _TPW
    for c in range(_TPW // _CCH):
        tb = base + c * _CCH
        pltpu.sync_copy(pos_hbm.at[0, pl.ds(tb, _CCH)], i1_v)
        pltpu.sync_copy(pos_hbm.at[1, pl.ds(tb, _CCH)], i2_v)
        cp1 = pltpu.async_copy(ys_hbm.at[i1_v], r1_v, sem1)
        cp2 = pltpu.async_copy(ys_hbm.at[i2_v], r2_v, sem2)
        cp1.wait()
        cp2.wait()

        def addt(t, carry):
            for k in range(_D // 16):
                sl = pl.ds(k * 16, 16)
                r1_v[t, sl] = r1_v[t, sl] + r2_v[t, sl]
            return carry

        lax.fori_loop(0, _CCH, addt, 0)
        pltpu.sync_copy(r1_v, y_hbm.at[pl.ds(tb, _CCH)])


@functools.cache
def _sc_kernels():
    mesh = plsc.VectorSubcoreMesh(core_axis_name="c", subcore_axis_name="s")
    no_layout = pltpu.CompilerParams(needs_layout_passes=False)
    scatter = pl.kernel(
        _scatter_body,
        compiler_params=no_layout,
        out_type=(
            jax.ShapeDtypeStruct((_PAD,), jnp.int32),
            jax.ShapeDtypeStruct((_PAD,), jnp.float32),
        ),
        mesh=mesh,
        scratch_types=[
            pltpu.VMEM((2 * _S,), jnp.int32),
            pltpu.VMEM((2 * _S,), jnp.float32),
            pltpu.VMEM((_PAD,), jnp.int32),
            pltpu.VMEM((_PAD,), jnp.float32),
        ],
    )
    gather = pl.kernel(
        _gather_body,
        out_type=jax.ShapeDtypeStruct((_PAD, _D), jnp.float32),
        mesh=mesh,
        scratch_types=[
            pltpu.VMEM((_GCH,), jnp.int32),
            pltpu.VMEM((_GCH,), jnp.int32),
            pltpu.VMEM((_GCH, _D), jnp.float32),
            pltpu.VMEM((_GCH, _D), jnp.float32),
            pltpu.SemaphoreType.DMA,
            pltpu.SemaphoreType.DMA,
        ],
    )
    combine = pl.kernel(
        _combine_body,
        out_type=jax.ShapeDtypeStruct((_S, _D), jnp.float32),
        mesh=mesh,
        scratch_types=[
            pltpu.VMEM((_CCH,), jnp.int32),
            pltpu.VMEM((_CCH,), jnp.int32),
            pltpu.VMEM((_CCH, _D), jnp.float32),
            pltpu.VMEM((_CCH, _D), jnp.float32),
            pltpu.SemaphoreType.DMA,
            pltpu.SemaphoreType.DMA,
        ],
    )
    return scatter, gather, combine


def kernel(x, gate_w, Wg, Wu, Wd):
    b, s, d = x.shape
    scatter, gather, combine = _sc_kernels()
    x2d = x.reshape(s, d)
    pos12, probs12, te = _router(x2d, gate_w)
    src, ps = scatter(pos12.reshape(2 * _S), probs12.reshape(2 * _S))
    xs = gather(src, x2d)
    ys = _grouped(te.reshape(_NT), xs, Wg, Wu, Wd, ps.reshape(_PAD, 1))
    y = combine(ys, pos12)
    return y.reshape(b, s, d)


# speedup vs baseline: 1.6171x; 1.5847x over previous
import functools

import jax
import jax.numpy as jnp
from jax import lax
from jax.experimental import pallas as pl
from jax.experimental.pallas import tpu as pltpu
from jax.experimental.pallas import tpu_sc as plsc

_E = 16
_D = 1024
_DFF = 512
_S = 2048
_TM = 128
_NT = 48
_PAD = _NT * _TM
_NW = 32
_RPW = _PAD // _NW
_TPW = _S // _NW


def _router_body(x_ref, gate_ref, pos_ref, prob_ref, te_ref):
    x = x_ref[:]
    scores = lax.dot_general(gate_ref[:], x, (((1,), (1,)), ((), ())),
                             preferred_element_type=jnp.float32)
    ie = lax.broadcasted_iota(jnp.int32, (_E, _S), 0)
    m1 = jnp.max(scores, axis=0, keepdims=True)
    i1 = jnp.min(jnp.where(scores == m1, ie, _E), axis=0, keepdims=True)
    masked = jnp.where(ie == i1, -jnp.inf, scores)
    m2 = jnp.max(masked, axis=0, keepdims=True)
    i2 = jnp.min(jnp.where(masked == m2, ie, _E), axis=0, keepdims=True)
    t = jnp.exp(m2 - m1)
    p1 = 1.0 / (1.0 + t)
    p2 = 1.0 - p1
    maskf = jnp.logical_or(ie == i1, ie == i2).astype(jnp.float32)

    lu = (lax.broadcasted_iota(jnp.int32, (_TM, _TM), 0)
          <= lax.broadcasted_iota(jnp.int32, (_TM, _TM), 1)).astype(jnp.float32)
    chunks = []
    carry = jnp.zeros((_E, 1), jnp.float32)
    for c in range(_S // _TM):
        mc = maskf[:, c * _TM:(c + 1) * _TM]
        cs = lax.dot_general(mc, lu, (((1,), (0,)), ((), ())),
                             preferred_element_type=jnp.float32)
        chunks.append(cs + carry)
        carry = carry + cs[:, _TM - 1:_TM]
    incl = jnp.concatenate(chunks, axis=1)

    cnt = incl[:, _S - 1:_S]
    cntp = jnp.ceil(cnt * (1.0 / _TM)) * float(_TM)
    slm = (lax.broadcasted_iota(jnp.int32, (_E, _E), 1)
           < lax.broadcasted_iota(jnp.int32, (_E, _E), 0)).astype(jnp.float32)
    offp = lax.dot_general(slm, cntp, (((1,), (0,)), ((), ())),
                           preferred_element_type=jnp.float32)
    posmat = offp + incl - 1.0
    on1 = (ie == i1).astype(jnp.float32)
    on2 = (ie == i2).astype(jnp.float32)
    pos1 = jnp.sum(on1 * posmat, axis=0, keepdims=True)
    pos2 = jnp.sum(on2 * posmat, axis=0, keepdims=True)
    pos_ref[:] = jnp.concatenate([pos1, pos2], axis=0).astype(jnp.int32)
    prob_ref[:] = jnp.concatenate([p1, p2], axis=0)

    ends = offp + cntp
    jv = (lax.broadcasted_iota(jnp.int32, (_E, _NT), 1) * _TM).astype(jnp.float32)
    tef = jnp.sum((ends <= jv).astype(jnp.float32), axis=0, keepdims=True)
    te_ref[:] = jnp.minimum(tef, float(_E - 1)).astype(jnp.int32)


def _router(x2d, gate_w, interpret=False):
    return pl.pallas_call(
        _router_body,
        out_shape=(
            jax.ShapeDtypeStruct((2, _S), jnp.int32),
            jax.ShapeDtypeStruct((2, _S), jnp.float32),
            jax.ShapeDtypeStruct((1, _NT), jnp.int32),
        ),
        interpret=interpret,
    )(x2d, gate_w)


def _scatter_body(pos_hbm, prob_hbm, src_hbm, ps_hbm, pos_v, prob_v, srcb, psb):
    cid = lax.axis_index("c")
    sid = lax.axis_index("s")

    @pl.when(jnp.logical_and(cid == 0, sid == 0))
    def _():
        pltpu.sync_copy(pos_hbm, pos_v)
        pltpu.sync_copy(prob_hbm, prob_v)
        lane = lax.iota(jnp.int32, 16)
        zf = jnp.zeros((16,), jnp.float32)

        def init(i, c):
            srcb[pl.ds(i * 16, 16)] = lane + lax.rem(i * 16, _S)
            psb[pl.ds(i * 16, 16)] = zf
            return c

        lax.fori_loop(0, _PAD // 16, init, 0)

        def body(i, c):
            base = i * 16
            p = pos_v[pl.ds(base, 16)]
            pr = prob_v[pl.ds(base, 16)]
            tok = lane + lax.rem(base, _S)
            plsc.store_scatter(srcb, [p], tok)
            plsc.store_scatter(psb, [p], pr)
            return c

        lax.fori_loop(0, (2 * _S) // 16, body, 0)
        pltpu.sync_copy(srcb, src_hbm)
        pltpu.sync_copy(psb, ps_hbm)


_GCH = 32
_GNB = 3


def _gather_body(src_hbm, x_hbm, xs_hbm, i1_v, i2_v, r1_v, r2_v, sem1, sem2):
    wid = lax.axis_index("s") * 2 + lax.axis_index("c")
    base = wid * _RPW
    for c in range(_RPW // (2 * _GCH)):
        cb = base + c * 2 * _GCH
        pltpu.sync_copy(src_hbm.at[pl.ds(cb, _GCH)], i1_v)
        pltpu.sync_copy(src_hbm.at[pl.ds(cb + _GCH, _GCH)], i2_v)
        cp1 = pltpu.async_copy(x_hbm.at[i1_v], r1_v, sem1)
        cp2 = pltpu.async_copy(x_hbm.at[i2_v], r2_v, sem2)
        cp1.wait()
        cp2.wait()
        pltpu.sync_copy(r1_v, xs_hbm.at[pl.ds(cb, _GCH)])
        pltpu.sync_copy(r2_v, xs_hbm.at[pl.ds(cb + _GCH, _GCH)])


def _grouped_body(te_ref, x_ref, wg_ref, wu_ref, wd_ref, pb_ref, o_ref):
    x = x_ref[:]
    g = lax.dot_general(x, wg_ref[0], (((1,), (1,)), ((), ())),
                        preferred_element_type=jnp.float32,
                        precision=lax.Precision.DEFAULT)
    u = lax.dot_general(x, wu_ref[0], (((1,), (1,)), ((), ())),
                        preferred_element_type=jnp.float32,
                        precision=lax.Precision.DEFAULT)
    h = (g * (1.0 / (1.0 + jnp.exp(-g)))) * u
    out = lax.dot_general(h, wd_ref[0], (((1,), (1,)), ((), ())),
                          preferred_element_type=jnp.float32,
                          precision=lax.Precision.DEFAULT)
    o_ref[:] = out * pb_ref[:]


def _grouped(te, xs, Wg, Wu, Wd, ps, interpret=False):
    grid_spec = pltpu.PrefetchScalarGridSpec(
        num_scalar_prefetch=1,
        grid=(_NT,),
        in_specs=[
            pl.BlockSpec((_TM, _D), lambda j, te: (j, 0)),
            pl.BlockSpec((1, _DFF, _D), lambda j, te: (te[j], 0, 0)),
            pl.BlockSpec((1, _DFF, _D), lambda j, te: (te[j], 0, 0)),
            pl.BlockSpec((1, _D, _DFF), lambda j, te: (te[j], 0, 0)),
            pl.BlockSpec((_TM, 1), lambda j, te: (j, 0)),
        ],
        out_specs=pl.BlockSpec((_TM, _D), lambda j, te: (j, 0)),
    )
    return pl.pallas_call(
        _grouped_body,
        grid_spec=grid_spec,
        out_shape=jax.ShapeDtypeStruct((_PAD, _D), jnp.float32),
        interpret=interpret,
    )(te, xs, Wg, Wu, Wd, ps)


_CCH = 32


def _combine_body(ys_hbm, pos_hbm, y_hbm, i1_v, i2_v, r1_v, r2_v, sem1, sem2):
    wid = lax.axis_index("s") * 2 + lax.axis_index("c")
    base = wid * _TPW
    for c in range(_TPW // _CCH):
        tb = base + c * _CCH
        pltpu.sync_copy(pos_hbm.at[0, pl.ds(tb, _CCH)], i1_v)
        pltpu.sync_copy(pos_hbm.at[1, pl.ds(tb, _CCH)], i2_v)
        cp1 = pltpu.async_copy(ys_hbm.at[i1_v], r1_v, sem1)
        cp2 = pltpu.async_copy(ys_hbm.at[i2_v], r2_v, sem2)
        cp1.wait()
        cp2.wait()

        def addt(t, carry):
            for k in range(_D // 16):
                sl = pl.ds(k * 16, 16)
                r1_v[t, sl] = r1_v[t, sl] + r2_v[t, sl]
            return carry

        lax.fori_loop(0, _CCH, addt, 0)
        pltpu.sync_copy(r1_v, y_hbm.at[pl.ds(tb, _CCH)])


@functools.cache
def _sc_kernels():
    mesh = plsc.VectorSubcoreMesh(core_axis_name="c", subcore_axis_name="s")
    no_layout = pltpu.CompilerParams(needs_layout_passes=False)
    scatter = pl.kernel(
        _scatter_body,
        compiler_params=no_layout,
        out_type=(
            jax.ShapeDtypeStruct((_PAD,), jnp.int32),
            jax.ShapeDtypeStruct((_PAD,), jnp.float32),
        ),
        mesh=mesh,
        scratch_types=[
            pltpu.VMEM((2 * _S,), jnp.int32),
            pltpu.VMEM((2 * _S,), jnp.float32),
            pltpu.VMEM((_PAD,), jnp.int32),
            pltpu.VMEM((_PAD,), jnp.float32),
        ],
    )
    gather = pl.kernel(
        _gather_body,
        out_type=jax.ShapeDtypeStruct((_PAD, _D), jnp.float32),
        mesh=mesh,
        scratch_types=[
            pltpu.VMEM((_GCH,), jnp.int32),
            pltpu.VMEM((_GCH,), jnp.int32),
            pltpu.VMEM((_GCH, _D), jnp.float32),
            pltpu.VMEM((_GCH, _D), jnp.float32),
            pltpu.SemaphoreType.DMA,
            pltpu.SemaphoreType.DMA,
        ],
    )
    combine = pl.kernel(
        _combine_body,
        out_type=jax.ShapeDtypeStruct((_S, _D), jnp.float32),
        mesh=mesh,
        scratch_types=[
            pltpu.VMEM((_CCH,), jnp.int32),
            pltpu.VMEM((_CCH,), jnp.int32),
            pltpu.VMEM((_CCH, _D), jnp.float32),
            pltpu.VMEM((_CCH, _D), jnp.float32),
            pltpu.SemaphoreType.DMA,
            pltpu.SemaphoreType.DMA,
        ],
    )
    return scatter, gather, combine


def kernel(x, gate_w, Wg, Wu, Wd):
    b, s, d = x.shape
    scatter, gather, combine = _sc_kernels()
    x2d = x.reshape(s, d)
    pos12, probs12, te = _router(x2d, gate_w)
    src, ps = scatter(pos12.reshape(2 * _S), probs12.reshape(2 * _S))
    xs = gather(src, x2d)
    ys = _grouped(te.reshape(_NT), xs, Wg, Wu, Wd, ps.reshape(_PAD, 1))
    y = combine(ys, pos12)
    return y.reshape(b, s, d)
